# Initial kernel scaffold; baseline (speedup 1.0000x reference)
import jax
import jax.numpy as jnp
from jax.experimental import pallas as pl


def kernel(x_node, edge_index_rel0, edge_index_rel1, edge_index_rel2, edge_index_rel3, edge_label_index, snap, W1l, W1r, b1, W2l, W2r, b2, Wa1, ba1, qa1, Wa2, ba2, qa2, Wp, bp):
    raise NotImplementedError("write your pallas kernel here")


# speedup vs baseline: 2.4194x; 2.4194x over previous
import functools

import jax
import jax.numpy as jnp
from jax import lax
from jax.experimental import pallas as pl
from jax.experimental.pallas import tpu as pltpu
from jax.experimental.pallas import tpu_sc as plsc

N = 10000
F = 256
H1 = 128
H2 = 64
R = 4
E = 40000
L = 20000

NC = 2
NS = 16

NP = 10240
EP = 40960
KCH = 20
CH = 128
PR = NP // NS
W1T = H1 + 16
W2T = H2 + 16
BN = 512
LP = 2 * L + 960
LK = 10
BL = 1000
NBL = L // BL


def _sc_mesh():
  return plsc.VectorSubcoreMesh(
      core_axis_name="c", subcore_axis_name="s", num_cores=NC, num_subcores=NS)


def _make_segsum(Wd):
  @functools.partial(
      pl.kernel,
      out_type=jax.ShapeDtypeStruct((R * NP, Wd), jnp.float32),
      mesh=_sc_mesh(),
      scratch_types=[
          pltpu.VMEM((KCH, CH), jnp.int32),
          pltpu.VMEM((KCH, CH), jnp.int32),
          pltpu.VMEM((CH, Wd), jnp.float32),
          pltpu.VMEM_SHARED((NP, Wd), jnp.float32),
          pltpu.SemaphoreType.DMA,
      ],
      compiler_params=pltpu.CompilerParams(use_tc_tiling_on_sc=False),
  )
  def segsum(table_hbm, srcs_hbm, dsts_hbm, zeros_hbm, out_hbm,
             src_v, dst_v, rows_v, acc_sh, sem):
    c = lax.axis_index("c")
    s = lax.axis_index("s")
    for rr in range(R // NC):
      r = rr * NC + c
      pltpu.sync_copy(zeros_hbm.at[pl.ds(s * PR, PR)],
                      acc_sh.at[pl.ds(s * PR, PR)])
      pltpu.sync_copy(srcs_hbm.at[r * NS + s], src_v)
      pltpu.sync_copy(dsts_hbm.at[r * NS + s], dst_v)
      plsc.subcore_barrier()

      def body(j, carry):
        pltpu.async_copy(table_hbm.at[src_v.at[j]], rows_v, sem).wait()
        pltpu.sync_copy(rows_v, acc_sh.at[dst_v.at[j]], add=True)
        return carry

      lax.fori_loop(0, KCH, body, 0)
      plsc.subcore_barrier()
      pltpu.sync_copy(acc_sh.at[pl.ds(s * PR, PR)],
                      out_hbm.at[pl.ds(r * NP + s * PR, PR)])
      plsc.subcore_barrier()

  return segsum


def _make_label_gather():
  @functools.partial(
      pl.kernel,
      out_type=jax.ShapeDtypeStruct((LP, H2), jnp.float32),
      mesh=_sc_mesh(),
      scratch_types=[
          pltpu.VMEM((LK, CH), jnp.int32),
          pltpu.VMEM((CH, H2), jnp.float32),
          pltpu.SemaphoreType.DMA,
      ],
      compiler_params=pltpu.CompilerParams(use_tc_tiling_on_sc=False),
  )
  def gather(h2_hbm, idx_hbm, out_hbm, idx_v, rows_v, sem):
    c = lax.axis_index("c")
    s = lax.axis_index("s")
    w = s * NC + c
    pltpu.sync_copy(idx_hbm.at[w], idx_v)

    def body(j, carry):
      pltpu.async_copy(h2_hbm.at[idx_v.at[j]], rows_v, sem).wait()
      pltpu.sync_copy(rows_v, out_hbm.at[pl.ds(w * (LK * CH) + j * CH, CH)])
      return carry

    lax.fori_loop(0, LK, body, 0)

  return gather


def _proj1_body(x_ref, wl_ref, wr_ref, b_ref, tbl_ref, xr_ref):
  xb = x_ref[...]
  y = jnp.dot(xb, wl_ref[...], preferred_element_type=jnp.float32)
  tbl_ref[:, :H1] = y
  col = lax.broadcasted_iota(jnp.int32, (BN, 16), 1)
  tbl_ref[:, H1:] = jnp.where(col == 0, 1.0, 0.0).astype(jnp.float32)
  xr_ref[...] = (jnp.dot(xb, wr_ref[...], preferred_element_type=jnp.float32)
                 + b_ref[...])


def _proj1(x_pad, W1l, W1r, b1):
  nb = NP // BN
  return pl.pallas_call(
      _proj1_body,
      grid=(R, nb),
      in_specs=[
          pl.BlockSpec((BN, F), lambda r, i: (i, 0)),
          pl.BlockSpec((None, F, H1), lambda r, i: (r, 0, 0)),
          pl.BlockSpec((None, F, H1), lambda r, i: (r, 0, 0)),
          pl.BlockSpec((None, 1, H1), lambda r, i: (r, 0, 0)),
      ],
      out_specs=[
          pl.BlockSpec((None, BN, W1T), lambda r, i: (r, i, 0)),
          pl.BlockSpec((None, BN, H1), lambda r, i: (r, i, 0)),
      ],
      out_shape=[
          jax.ShapeDtypeStruct((R, NP, W1T), jnp.float32),
          jax.ShapeDtypeStruct((R, NP, H1), jnp.float32),
      ],
  )(x_pad, W1l, W1r, b1.reshape(R, 1, H1))


def _make_att(D, Wd):
  def body(acc_ref, xr_ref, wa_ref, ba_ref, qa_ref, out_ref, ws_ref):
    i = pl.program_id(0)

    @pl.when(i == 0)
    def _():
      ws_ref[...] = jnp.zeros_like(ws_ref)

    rowid = i * BN + lax.broadcasted_iota(jnp.int32, (BN, 1), 0)
    mask = rowid < N
    wa = wa_ref[...]
    ba = ba_ref[...]
    qa = qa_ref[...]
    contribs = []
    for r in range(R):
      cnt = acc_ref[r, :, D:D + 1]
      o = acc_ref[r, :, :D] / jnp.maximum(cnt, 1.0) + xr_ref[r]
      out_ref[:, r, :] = o
      t = jnp.dot(jnp.tanh(jnp.dot(o, wa, preferred_element_type=jnp.float32)
                           + ba), qa, preferred_element_type=jnp.float32)
      contribs.append(jnp.sum(jnp.where(mask, t, 0.0)))
    ws_ref[...] += jnp.stack(contribs).reshape(1, R)

  def run(acc, xr, Wa, ba, qa):
    nb = NP // BN
    return pl.pallas_call(
        body,
        grid=(nb,),
        in_specs=[
            pl.BlockSpec((R, BN, Wd), lambda i: (0, i, 0)),
            pl.BlockSpec((R, BN, D), lambda i: (0, i, 0)),
            pl.BlockSpec((D, D), lambda i: (0, 0)),
            pl.BlockSpec((1, D), lambda i: (0, 0)),
            pl.BlockSpec((D, 1), lambda i: (0, 0)),
        ],
        out_specs=[
            pl.BlockSpec((BN, R, D), lambda i: (i, 0, 0)),
            pl.BlockSpec((1, R), lambda i: (0, 0)),
        ],
        out_shape=[
            jax.ShapeDtypeStruct((NP, R, D), jnp.float32),
            jax.ShapeDtypeStruct((1, R), jnp.float32),
        ],
    )(acc, xr, Wa, ba.reshape(1, D), qa)

  return run


def _softmax_row(w):
  m = jnp.max(w, axis=1, keepdims=True)
  e = jnp.exp(w - m)
  return e / jnp.sum(e, axis=1, keepdims=True)


def _combine2_body(o1_ref, ws_ref, w2l_ref, w2r_ref, b2_ref, tbl_ref, xr_ref):
  beta = _softmax_row(ws_ref[...] / float(N))
  blk = o1_ref[...]
  h1 = jnp.sum(blk * beta.reshape(1, R, 1), axis=1)
  col = lax.broadcasted_iota(jnp.int32, (BN, 16), 1)
  ones = jnp.where(col == 0, 1.0, 0.0).astype(jnp.float32)
  for r in range(R):
    tbl_ref[r, :, :H2] = jnp.dot(h1, w2l_ref[r],
                                 preferred_element_type=jnp.float32)
    tbl_ref[r, :, H2:] = ones
    xr_ref[r, :, :] = (jnp.dot(h1, w2r_ref[r],
                               preferred_element_type=jnp.float32)
                       + b2_ref[r])


def _combine2(out1p, wsum1, W2l, W2r, b2):
  nb = NP // BN
  return pl.pallas_call(
      _combine2_body,
      grid=(nb,),
      in_specs=[
          pl.BlockSpec((BN, R, H1), lambda i: (i, 0, 0)),
          pl.BlockSpec((1, R), lambda i: (0, 0)),
          pl.BlockSpec((R, H1, H2), lambda i: (0, 0, 0)),
          pl.BlockSpec((R, H1, H2), lambda i: (0, 0, 0)),
          pl.BlockSpec((R, 1, H2), lambda i: (0, 0, 0)),
      ],
      out_specs=[
          pl.BlockSpec((R, BN, W2T), lambda i: (0, i, 0)),
          pl.BlockSpec((R, BN, H2), lambda i: (0, i, 0)),
      ],
      out_shape=[
          jax.ShapeDtypeStruct((R, NP, W2T), jnp.float32),
          jax.ShapeDtypeStruct((R, NP, H2), jnp.float32),
      ],
  )(out1p, wsum1, W2l, W2r, b2.reshape(R, 1, H2))


def _h2_body(o2_ref, ws_ref, h2_ref):
  beta = _softmax_row(ws_ref[...] / float(N))
  h2_ref[...] = jnp.sum(o2_ref[...] * beta.reshape(1, R, 1), axis=1)


def _h2_combine(out2p, wsum2):
  nb = NP // BN
  return pl.pallas_call(
      _h2_body,
      grid=(nb,),
      in_specs=[
          pl.BlockSpec((BN, R, H2), lambda i: (i, 0, 0)),
          pl.BlockSpec((1, R), lambda i: (0, 0)),
      ],
      out_specs=pl.BlockSpec((BN, H2), lambda i: (i, 0)),
      out_shape=jax.ShapeDtypeStruct((NP, H2), jnp.float32),
  )(out2p, wsum2)


def _scores_body(hs_ref, hd_ref, wp_ref, bp_ref, out_ref):
  had = hs_ref[...] * hd_ref[...]
  sc = jnp.dot(had, wp_ref[...], preferred_element_type=jnp.float32) + bp_ref[...]
  out_ref[...] = jnp.sum(sc, axis=-1).reshape(1, BL)


def _scores(hs, hd, Wp, bp):
  return pl.pallas_call(
      _scores_body,
      grid=(NBL,),
      in_specs=[
          pl.BlockSpec((BL, H2), lambda i: (i, 0)),
          pl.BlockSpec((BL, H2), lambda i: (i, 0)),
          pl.BlockSpec((H2, 2), lambda i: (0, 0)),
          pl.BlockSpec((1, 2), lambda i: (0, 0)),
      ],
      out_specs=pl.BlockSpec((None, 1, BL), lambda i: (i, 0, 0)),
      out_shape=jax.ShapeDtypeStruct((NBL, 1, BL), jnp.float32),
  )(hs, hd, Wp, bp.reshape(1, 2))


_att1 = _make_att(H1, W1T)
_att2 = _make_att(H2, W2T)


@functools.lru_cache(maxsize=None)
def _get_segsum(Wd):
  return _make_segsum(Wd)


@functools.lru_cache(maxsize=None)
def _get_lgather():
  return _make_label_gather()


def kernel(x_node, edge_index_rel0, edge_index_rel1, edge_index_rel2,
           edge_index_rel3, edge_label_index, snap, W1l, W1r, b1, W2l, W2r,
           b2, Wa1, ba1, qa1, Wa2, ba2, qa2, Wp, bp):
  x_pad = jnp.pad(x_node, ((0, NP - N), (0, 0)))

  eis = [edge_index_rel0, edge_index_rel1, edge_index_rel2, edge_index_rel3]
  srcs = jnp.stack([e[0] for e in eis]).astype(jnp.int32)
  dsts = jnp.stack([e[1] for e in eis]).astype(jnp.int32)
  srcs = jnp.pad(srcs, ((0, 0), (0, EP - E)), constant_values=N)
  dsts = jnp.pad(dsts, ((0, 0), (0, EP - E)), constant_values=N)
  srcs = srcs + (jnp.arange(R, dtype=jnp.int32) * NP)[:, None]
  srcs = srcs.reshape(R * NS, KCH, CH)
  dsts = dsts.reshape(R * NS, KCH, CH)

  zeros1 = jnp.zeros((NP, W1T), jnp.float32)
  zeros2 = jnp.zeros((NP, W2T), jnp.float32)

  table1, xr1 = _proj1(x_pad, W1l, W1r, b1)
  acc1 = _get_segsum(W1T)(table1.reshape(R * NP, W1T), srcs, dsts, zeros1)
  out1p, wsum1 = _att1(acc1.reshape(R, NP, W1T), xr1, Wa1, ba1, qa1)

  table2, xr2 = _combine2(out1p, wsum1, W2l, W2r, b2)
  acc2 = _get_segsum(W2T)(table2.reshape(R * NP, W2T), srcs, dsts, zeros2)
  out2p, wsum2 = _att2(acc2.reshape(R, NP, W2T), xr2, Wa2, ba2, qa2)

  h2 = _h2_combine(out2p, wsum2)
  lidx = jnp.concatenate([edge_label_index[0], edge_label_index[1]])
  lidx = jnp.pad(lidx.astype(jnp.int32), (0, LP - 2 * L))
  rows = _get_lgather()(h2, lidx.reshape(NC * NS, LK, CH))
  hs = rows[:L]
  hd = rows[L:2 * L]
  scores = _scores(hs, hd, Wp, bp).reshape(L)

  return (scores, out1p[:N], out2p[:N])
